# baseline (speedup 1.0000x reference)
import functools

import jax
import jax.numpy as jnp
from jax import lax
from jax.experimental import pallas as pl
from jax.experimental.pallas import tpu as pltpu
from jax.experimental.pallas import tpu_sc as plsc


PAD = 128


@functools.lru_cache(maxsize=None)
def _make_lookup(batch, seq, embed, nc, ns, nb):
    nw = nc * ns
    rows_per_w = batch // nw
    n_chunks = rows_per_w // nb
    assert batch % nw == 0 and rows_per_w % nb == 0
    assert n_chunks >= 4 and n_chunks % 2 == 0
    splits = []
    off = 0
    while off < seq:
        width = min(128, seq - off)
        splits.append((off, width))
        off += width
    mesh = plsc.VectorSubcoreMesh(core_axis_name="c", subcore_axis_name="s")

    @functools.partial(
        pl.kernel,
        out_type=jax.ShapeDtypeStruct((batch, seq, embed), jnp.float32),
        mesh=mesh,
        scratch_types=[
            pltpu.VMEM((2, nb, seq), jnp.int32),
            pltpu.VMEM((2, nb, seq, PAD), jnp.float32),
            pltpu.SemaphoreType.DMA,
            pltpu.SemaphoreType.DMA,
            pltpu.SemaphoreType.DMA,
            pltpu.SemaphoreType.DMA,
        ],
        compiler_params=pltpu.CompilerParams(use_tc_tiling_on_sc=False),
    )
    def lookup(idx_hbm, table_hbm, out_hbm, idx_v, rows_v, sg0, sg1, sw0, sw1):
        wid = lax.axis_index("s") * nc + lax.axis_index("c")
        b_base = wid * rows_per_w
        sg = (sg0, sg1)
        sw = (sw0, sw1)

        def load_idx(c, p):
            pltpu.sync_copy(idx_hbm.at[pl.ds(b_base + c * nb, nb)], idx_v.at[p])

        def fire_gathers(p):
            for j in range(nb):
                for off, width in splits:
                    pltpu.async_copy(
                        table_hbm.at[idx_v.at[p].at[j, pl.ds(off, width)]],
                        rows_v.at[p].at[j, pl.ds(off, width)],
                        sg[p],
                    )

        def drain_gathers(p):
            for j in range(nb):
                for off, width in splits:
                    pltpu.make_async_copy(
                        table_hbm.at[idx_v.at[p].at[j, pl.ds(off, width)]],
                        rows_v.at[p].at[j, pl.ds(off, width)],
                        sg[p],
                    ).wait()

        def fire_write(c, p):
            return pltpu.async_copy(
                rows_v.at[p].at[:, :, pl.ds(0, embed)],
                out_hbm.at[pl.ds(b_base + c * nb, nb)],
                sw[p],
            )

        load_idx(0, 0)
        fire_gathers(0)
        load_idx(1, 1)
        fire_gathers(1)

        def superstep(s, _):
            for p in range(2):
                c = 2 * s + p
                drain_gathers(p)
                w = fire_write(c, p)
                load_idx(c + 2, p)
                w.wait()
                fire_gathers(p)
            return 0

        lax.fori_loop(0, (n_chunks - 2) // 2, superstep, 0)

        for p in range(2):
            c = n_chunks - 2 + p
            drain_gathers(p)
            fire_write(c, p).wait()

    return lookup


def kernel(inputs, embedding_table):
    b, s = inputs.shape
    v, e = embedding_table.shape
    table_p = jnp.pad(embedding_table, ((0, 0), (0, 128 - e)))
    info = plsc.get_sparse_core_info()
    lookup = _make_lookup(b, s, e, info.num_cores, info.num_subcores, 2)
    return lookup(inputs, table_p)
